# baseline (speedup 1.0000x reference)
import functools

import numpy as np
import jax
import jax.numpy as jnp
from jax import lax
from jax.experimental import pallas as pl
from jax.experimental.pallas import tpu as pltpu, tpu_sc as plsc

_NC = 2
_NS = 16
_NW = _NC * _NS
_CR = 128


def _pos_tables(seq_len, d):
    j = np.arange(d // 2, dtype=np.float64)
    units = 10000.0 ** (2.0 * j / d)
    p = np.arange(seq_len, dtype=np.float64)[:, None]
    angle = p / units[None, :]
    pos = np.zeros((seq_len, d), dtype=np.float64)
    pos[:, 0::2] = np.sin(angle)
    pos[:, 1::2] = np.cos(angle)
    w = 4 * ((2 * seq_len + 15 + 127) // 128) * 32
    post2 = np.zeros((d, w), dtype=np.float64)
    post2[:, :seq_len] = pos.T
    post2[:, seq_len:2 * seq_len] = pos.T
    return jnp.asarray(post2, dtype=jnp.float32)


@functools.lru_cache(maxsize=None)
def _make_sc_kernel(B, S, D):
    n_row = B * S
    n_row_w = n_row // _NW
    n_chunk = n_row_w // _CR
    assert n_row % (_NW * _CR) == 0 and D == 64 and S % 8 == 0
    n_grp = _CR // 16
    pos_w = 4 * ((2 * S + 15 + 127) // 128) * 32
    mesh = plsc.VectorSubcoreMesh(core_axis_name="c", subcore_axis_name="s")

    @functools.partial(
        pl.kernel,
        out_type=jax.ShapeDtypeStruct((n_row, D), jnp.float32),
        mesh=mesh,
        compiler_params=pltpu.CompilerParams(needs_layout_passes=False),
        scratch_types=[
            pltpu.VMEM((n_row_w,), jnp.int32),
            pltpu.VMEM((2, _CR), jnp.int32),
            pltpu.VMEM((2 * _CR, 2 * D), jnp.float32),
            pltpu.VMEM((2 * _CR, D), jnp.float32),
            pltpu.VMEM((D, pos_w), jnp.float32),
            pltpu.SemaphoreType.DMA,
            pltpu.SemaphoreType.DMA,
        ],
    )
    def k(x_hbm, table_hbm, pos_hbm, out_hbm, idx_all, idx_g, big_v,
          out_v, pos_v, sem_g, sem_s):
        wid = lax.axis_index("s") * _NC + lax.axis_index("c")
        row0 = wid * n_row_w
        pltpu.sync_copy(x_hbm.at[pl.ds(row0, n_row_w)], idx_all)
        pltpu.sync_copy(pos_hbm, pos_v)
        lane = jax.lax.iota(jnp.int32, 16)

        def prep_and_fire(c, b):
            @plsc.parallel_loop(0, _CR // 16, unroll=4)
            def _(g):
                idx16 = idx_all[pl.ds(c * _CR + g * 16, 16)]
                idx_g[b, pl.ds(g * 16, 16)] = lax.shift_right_logical(idx16, 1)
            pltpu.async_copy(
                table_hbm.at[idx_g.at[b]],
                big_v.at[pl.ds(b * _CR, _CR)],
                sem_g,
            )

        def wait_gather():
            pltpu.make_async_copy(
                table_hbm.at[idx_g.at[0]],
                big_v.at[pl.ds(0, _CR)],
                sem_g,
            ).wait()

        def select_add(c, b):
            for g in range(n_grp):
                base = c * _CR + g * 16
                s0 = lax.rem(row0 + base, S)
                row16 = b * _CR + g * 16 + lane
                idx16 = idx_all[pl.ds(base, 16)]
                half = lax.shift_left(
                    lax.bitwise_and(idx16, jnp.int32(1)), 6)

                s16 = s0 + lane

                @plsc.parallel_loop(0, D, unroll=8)
                def _(d):
                    d16 = jnp.broadcast_to(d, (16,))
                    v = plsc.load_gather(big_v, [row16, half + d])
                    p = plsc.load_gather(pos_v, [d16, s16])
                    plsc.store_scatter(out_v, [row16, d16], v + p)

        def fire_store(c, b):
            pltpu.async_copy(
                out_v.at[pl.ds(b * _CR, _CR)],
                out_hbm.at[pl.ds(row0 + c * _CR, _CR)],
                sem_s,
            )

        def wait_store():
            pltpu.make_async_copy(
                out_v.at[pl.ds(0, _CR)],
                out_hbm.at[pl.ds(0, _CR)],
                sem_s,
            ).wait()

        def body(c, _):
            b = c % 2
            @pl.when(c >= 2)
            def _():
                wait_store()

            @pl.when(c < n_chunk)
            def _():
                prep_and_fire(c, b)

            @pl.when(c >= 1)
            def _():
                wait_gather()
                select_add(c - 1, 1 - b)
                fire_store(c - 1, 1 - b)

            return 0

        lax.fori_loop(0, n_chunk + 1, body, 0)
        wait_store()

    return k


def kernel(x, table):
    B, S = x.shape
    D = table.shape[1]
    pos2 = _pos_tables(S, D)
    idx = x.astype(jnp.int32).reshape(B * S)
    table2 = table.reshape(table.shape[0] // 2, 2 * D)
    out = _make_sc_kernel(B, S, D)(idx, table2, pos2)
    return out.reshape(B, S, D)


# speedup vs baseline: 1.0339x; 1.0339x over previous
import functools

import numpy as np
import jax
import jax.numpy as jnp
from jax import lax
from jax.experimental import pallas as pl
from jax.experimental.pallas import tpu as pltpu, tpu_sc as plsc

_NC = 2
_NS = 16
_NW = _NC * _NS
_CR = 128


def _pos_tables(seq_len, d):
    j = np.arange(d // 2, dtype=np.float64)
    units = 10000.0 ** (2.0 * j / d)
    p = np.arange(seq_len, dtype=np.float64)[:, None]
    angle = p / units[None, :]
    pos = np.zeros((seq_len, d), dtype=np.float64)
    pos[:, 0::2] = np.sin(angle)
    pos[:, 1::2] = np.cos(angle)
    w = 4 * ((2 * seq_len + 15 + 127) // 128) * 32
    post2 = np.zeros((d, w), dtype=np.float64)
    post2[:, :seq_len] = pos.T
    post2[:, seq_len:2 * seq_len] = pos.T
    return jnp.asarray(post2, dtype=jnp.float32)


@functools.lru_cache(maxsize=None)
def _make_sc_kernel(B, S, D):
    n_row = B * S
    n_row_w = n_row // _NW
    n_chunk = n_row_w // _CR
    assert n_row % (_NW * _CR) == 0 and D == 64 and S % 8 == 0
    n_grp = _CR // 16
    pos_w = 4 * ((2 * S + 15 + 127) // 128) * 32
    mesh = plsc.VectorSubcoreMesh(core_axis_name="c", subcore_axis_name="s")

    @functools.partial(
        pl.kernel,
        out_type=jax.ShapeDtypeStruct((n_row, D), jnp.float32),
        mesh=mesh,
        compiler_params=pltpu.CompilerParams(needs_layout_passes=False),
        scratch_types=[
            pltpu.VMEM((n_row_w,), jnp.int32),
            pltpu.VMEM((2, _CR), jnp.int32),
            pltpu.VMEM((2 * _CR, 2 * D), jnp.float32),
            pltpu.VMEM((2 * _CR, D), jnp.float32),
            pltpu.VMEM((D, pos_w), jnp.float32),
            pltpu.SemaphoreType.DMA,
            pltpu.SemaphoreType.DMA,
        ],
    )
    def k(x_hbm, table_hbm, pos_hbm, out_hbm, idx_all, idx_g, big_v,
          out_v, pos_v, sem_g, sem_s):
        wid = lax.axis_index("s") * _NC + lax.axis_index("c")
        row0 = wid * n_row_w
        pltpu.sync_copy(x_hbm.at[pl.ds(row0, n_row_w)], idx_all)
        pltpu.sync_copy(pos_hbm, pos_v)
        lane = jax.lax.iota(jnp.int32, 16)

        def prep_and_fire(c, b):
            @plsc.parallel_loop(0, _CR // 16, unroll=4)
            def _(g):
                idx16 = idx_all[pl.ds(c * _CR + g * 16, 16)]
                idx_g[b, pl.ds(g * 16, 16)] = lax.shift_right_logical(idx16, 1)
            pltpu.async_copy(
                table_hbm.at[idx_g.at[b]],
                big_v.at[pl.ds(b * _CR, _CR)],
                sem_g,
            )

        def wait_gather():
            pltpu.make_async_copy(
                table_hbm.at[idx_g.at[0]],
                big_v.at[pl.ds(0, _CR)],
                sem_g,
            ).wait()

        def select_add(c, b):
            for g in range(n_grp):
                base = c * _CR + g * 16
                s0 = lax.rem(row0 + base, S)
                row16 = b * _CR + g * 16 + lane
                idx16 = idx_all[pl.ds(base, 16)]
                half = lax.shift_left(
                    lax.bitwise_and(idx16, jnp.int32(1)), 6)

                @plsc.parallel_loop(0, D, unroll=16)
                def _(d):
                    v = plsc.load_gather(big_v, [row16, half + d])
                    p = pos_v[d, pl.ds(s0, 16)]
                    plsc.store_scatter(
                        out_v, [row16, jnp.broadcast_to(d, (16,))], v + p)

        def fire_store(c, b):
            pltpu.async_copy(
                out_v.at[pl.ds(b * _CR, _CR)],
                out_hbm.at[pl.ds(row0 + c * _CR, _CR)],
                sem_s,
            )

        def wait_store():
            pltpu.make_async_copy(
                out_v.at[pl.ds(0, _CR)],
                out_hbm.at[pl.ds(0, _CR)],
                sem_s,
            ).wait()

        def body(c, _):
            b = c % 2
            @pl.when(c >= 2)
            def _():
                wait_store()

            @pl.when(c < n_chunk)
            def _():
                prep_and_fire(c, b)

            @pl.when(c >= 1)
            def _():
                wait_gather()
                select_add(c - 1, 1 - b)
                fire_store(c - 1, 1 - b)

            return 0

        lax.fori_loop(0, n_chunk + 1, body, 0)
        wait_store()

    return k


def kernel(x, table):
    B, S = x.shape
    D = table.shape[1]
    pos2 = _pos_tables(S, D)
    idx = x.astype(jnp.int32).reshape(B * S)
    table2 = table.reshape(table.shape[0] // 2, 2 * D)
    out = _make_sc_kernel(B, S, D)(idx, table2, pos2)
    return out.reshape(B, S, D)


# speedup vs baseline: 1.6484x; 1.5944x over previous
import functools

import numpy as np
import jax
import jax.numpy as jnp
from jax import lax
from jax.experimental import pallas as pl
from jax.experimental.pallas import tpu as pltpu, tpu_sc as plsc

_NC = 2
_NS = 16
_NW = _NC * _NS
_CH = 2


def _pos_table(seq_len, d):
    j = np.arange(d // 2, dtype=np.float64)
    units = 10000.0 ** (2.0 * j / d)
    p = np.arange(seq_len, dtype=np.float64)[:, None]
    angle = p / units[None, :]
    pos = np.zeros((seq_len, d), dtype=np.float64)
    pos[:, 0::2] = np.sin(angle)
    pos[:, 1::2] = np.cos(angle)
    return jnp.asarray(pos, dtype=jnp.float32)


@functools.lru_cache(maxsize=None)
def _make_sc_kernel(B, S, D):
    assert B % (_NW * _CH) == 0 and D % 16 == 0 and S % 8 == 0
    n_seq_w = B // _NW
    n_chunk = n_seq_w // _CH
    g0 = min(128, S) // 8 * 8
    pieces = [(0, g0)]
    if g0 < S:
        pieces.append((g0, S - g0))
    mesh = plsc.VectorSubcoreMesh(core_axis_name="c", subcore_axis_name="s")

    @functools.partial(
        pl.kernel,
        out_type=jax.ShapeDtypeStruct((B, S, D), jnp.float32),
        mesh=mesh,
        compiler_params=pltpu.CompilerParams(use_tc_tiling_on_sc=False),
        scratch_types=[
            pltpu.VMEM((n_seq_w, S), jnp.int32),
            pltpu.VMEM((2, _CH, S, D), jnp.float32),
            pltpu.VMEM((S, D), jnp.float32),
            pltpu.SemaphoreType.DMA,
            pltpu.SemaphoreType.DMA,
        ],
    )
    def k(x_hbm, table_hbm, pos_hbm, out_hbm, idx_v, rows_v, pos_v,
          sem_g, sem_s):
        wid = lax.axis_index("s") * _NC + lax.axis_index("c")
        seq0 = wid * n_seq_w
        pltpu.sync_copy(x_hbm.at[pl.ds(seq0, n_seq_w)], idx_v)
        pltpu.sync_copy(pos_hbm, pos_v)

        def fire_gather(c, b):
            for s_off in range(_CH):
                for o, n in pieces:
                    pltpu.async_copy(
                        table_hbm.at[idx_v.at[c * _CH + s_off, pl.ds(o, n)]],
                        rows_v.at[b, s_off, pl.ds(o, n)],
                        sem_g,
                    )

        def wait_gather():
            for s_off in range(_CH):
                for o, n in pieces:
                    pltpu.make_async_copy(
                        table_hbm.at[idx_v.at[0, pl.ds(o, n)]],
                        rows_v.at[0, s_off, pl.ds(o, n)],
                        sem_g,
                    ).wait()

        def add_pos(b):
            for s_off in range(_CH):
                @plsc.parallel_loop(0, S, unroll=2)
                def _(r):
                    for dd in range(D // 16):
                        sl = pl.ds(dd * 16, 16)
                        rows_v[b, s_off, r, sl] = (
                            rows_v[b, s_off, r, sl] + pos_v[r, sl]
                        )

        def fire_store(c, b):
            pltpu.async_copy(
                rows_v.at[b],
                out_hbm.at[pl.ds(seq0 + c * _CH, _CH)],
                sem_s,
            )

        def wait_store():
            pltpu.make_async_copy(
                rows_v.at[0],
                out_hbm.at[pl.ds(0, _CH)],
                sem_s,
            ).wait()

        def body(c, _):
            b = c % 2
            @pl.when(c >= 2)
            def _():
                wait_store()

            @pl.when(c < n_chunk)
            def _():
                fire_gather(c, b)

            @pl.when(c >= 1)
            def _():
                wait_gather()
                add_pos(1 - b)
                fire_store(c - 1, 1 - b)

            return 0

        lax.fori_loop(0, n_chunk + 1, body, 0)
        wait_store()

    return k


def kernel(x, table):
    B, S = x.shape
    D = table.shape[1]
    pos = _pos_table(S, D)
    return _make_sc_kernel(B, S, D)(x.astype(jnp.int32), table, pos)
